# Initial kernel scaffold; baseline (speedup 1.0000x reference)
import jax
import jax.numpy as jnp
from jax.experimental import pallas as pl


def kernel(z, embeddings):
    raise NotImplementedError("write your pallas kernel here")


# speedup vs baseline: 3.2622x; 3.2622x over previous
import functools

import jax
import jax.numpy as jnp
from jax import lax
from jax.experimental import pallas as pl

NUM_EMBEDDINGS = 8192
EMBEDDING_DIM = 32
B, C, H, W = 8, 32, 32, 32
HW = H * W
N_TOTAL = B * HW * C

HW_T = 512
TK = 1024


def _vq_body(z_ref, emb_ref, q_ref, loss_ref):
    zb = z_ref[0]
    z2 = jnp.sum(zb * zb, axis=0, keepdims=True)

    best_val = jnp.full((1, HW_T), jnp.inf, dtype=jnp.float32)
    best_q = jnp.zeros((EMBEDDING_DIM, HW_T), dtype=jnp.float32)

    for kc in range(NUM_EMBEDDINGS // TK):
        e = emb_ref[kc * TK:(kc + 1) * TK, :]
        e2 = jnp.sum(e * e, axis=1, keepdims=True)
        m = lax.dot_general(e, zb, (((1,), (0,)), ((), ())),
                            preferred_element_type=jnp.float32)
        d = (z2 - 2.0 * m) + e2
        lmin = jnp.min(d, axis=0, keepdims=True)
        kio = lax.broadcasted_iota(jnp.int32, (TK, HW_T), 0)
        larg = jnp.min(jnp.where(d == lmin, kio, TK), axis=0, keepdims=True)
        onehot = (kio == larg).astype(jnp.float32)
        qc = lax.dot_general(e, onehot, (((0,), (0,)), ((), ())),
                             preferred_element_type=jnp.float32)
        upd = lmin < best_val
        best_val = jnp.where(upd, lmin, best_val)
        best_q = jnp.where(upd, qc, best_q)

    diff = best_q - zb
    q_ref[0] = zb + diff
    part = jnp.sum(diff * diff, axis=(0, 1), keepdims=True)

    @pl.when((pl.program_id(0) == 0) & (pl.program_id(1) == 0))
    def _init():
        loss_ref[...] = jnp.zeros((1, 1), jnp.float32)

    loss_ref[...] += part


@jax.jit
def kernel(z, embeddings):
    z3 = z.reshape(B, C, HW)
    grid = (B, HW // HW_T)
    q3, loss_acc = pl.pallas_call(
        _vq_body,
        grid=grid,
        in_specs=[
            pl.BlockSpec((1, C, HW_T), lambda b, h: (b, 0, h)),
            pl.BlockSpec((NUM_EMBEDDINGS, EMBEDDING_DIM), lambda b, h: (0, 0)),
        ],
        out_specs=[
            pl.BlockSpec((1, C, HW_T), lambda b, h: (b, 0, h)),
            pl.BlockSpec((1, 1), lambda b, h: (0, 0)),
        ],
        out_shape=[
            jax.ShapeDtypeStruct((B, C, HW), jnp.float32),
            jax.ShapeDtypeStruct((1, 1), jnp.float32),
        ],
    )(z3, embeddings)
    loss = loss_acc[0, 0] * (1.25 / N_TOTAL)
    return q3.reshape(B, C, H, W), loss
